# baseline (speedup 1.0000x reference)
import functools

import jax
import jax.numpy as jnp
from jax import lax
from jax.experimental import pallas as pl
from jax.experimental.pallas import tpu as pltpu
from jax.experimental.pallas import tpu_sc as plsc

N = 10000
E = 160000
F = 256
G = 64
OUT = 24
FH = 128
NTILE = 16
NP = 10112
RPT = NP // NTILE
CH = 64
CHE = 64
ECH = 168
EPAD = NTILE * ECH * CHE
GRPC = 24
NB = 3
PCH = 10
PPAD = NTILE * PCH * CH
GP = 128
GRPT = GP // NTILE
RB = 2528

NACC = NP + GP
CRPT = NACC // NTILE
CCH = 89
CPAD = 32 * CCH * CH


def _make_agg_deep(nch, ch, acc_rows, out_rpt):
    mesh = plsc.VectorSubcoreMesh(core_axis_name="c", subcore_axis_name="s")
    out_sds = jax.ShapeDtypeStruct((acc_rows, FH), jnp.float32)
    ngrp = nch // GRPC

    @functools.partial(
        pl.kernel,
        out_type=[out_sds, out_sds],
        mesh=mesh,
        scratch_types=[
            pltpu.VMEM((GRPC, ch), jnp.int32),
            pltpu.VMEM((GRPC, ch), jnp.int32),
        ] + [pltpu.VMEM((ch, FH), jnp.float32)] * NB
        + [pltpu.VMEM_SHARED((acc_rows, FH), jnp.float32)]
        + [pltpu.SemaphoreType.DMA] * (2 * NB),
    )
    def agg(h0, h1, srcs, dsts, zeros, o0, o1,
            src_v, dst_v, r0, r1, r2, acc,
            g0, g1, g2, s0, s1, s2):
        rows = [r0, r1, r2]
        gsem = [g0, g1, g2]
        ssem = [s0, s1, s2]
        c = lax.axis_index("c")
        s = lax.axis_index("s")
        pltpu.sync_copy(zeros.at[pl.ds(0, out_rpt)],
                        acc.at[pl.ds(s * out_rpt, out_rpt)])
        plsc.subcore_barrier()

        def run(h, o):
            def gcopy(j, b):
                return pltpu.make_async_copy(h.at[src_v.at[j]], rows[b],
                                             gsem[b])

            def scopy(j, b):
                return pltpu.make_async_copy(rows[b], acc.at[dst_v.at[j]],
                                             ssem[b])

            for g in range(ngrp):
                pltpu.sync_copy(srcs.at[s, pl.ds(g * GRPC, GRPC)], src_v)
                pltpu.sync_copy(dsts.at[s, pl.ds(g * GRPC, GRPC)], dst_v)
                gcopy(0, 0).start()
                gcopy(1, 1).start()

                def body(i, carry):
                    for b in range(NB):
                        j = i * NB + b
                        gcopy(j, b).wait()
                        scopy(j, b).start(add=True)
                        bn = (b + 2) % NB

                        @pl.when((j + 2 < GRPC) & (j >= 1))
                        def _():
                            scopy(j, bn).wait()

                        @pl.when(j + 2 < GRPC)
                        def _():
                            gcopy(j + 2, bn).start()

                    return carry

                lax.fori_loop(0, GRPC // NB, body, 0)
                for b in range(NB):
                    scopy(0, b).wait()

            plsc.subcore_barrier()
            pltpu.sync_copy(acc.at[pl.ds(s * out_rpt, out_rpt)],
                            o.at[pl.ds(s * out_rpt, out_rpt)])

        @pl.when(c == 0)
        def _():
            run(h0, o0)

        @pl.when(c == 1)
        def _():
            run(h1, o1)

    return agg


def _make_agg(nch, ch, acc_rows, out_rpt):
    mesh = plsc.VectorSubcoreMesh(core_axis_name="c", subcore_axis_name="s")
    out_sds = jax.ShapeDtypeStruct((acc_rows, FH), jnp.float32)

    @functools.partial(
        pl.kernel,
        out_type=[out_sds, out_sds],
        mesh=mesh,
        scratch_types=[
            pltpu.VMEM((nch, ch), jnp.int32),
            pltpu.VMEM((nch, ch), jnp.int32),
            pltpu.VMEM((ch, FH), jnp.float32),
            pltpu.VMEM((ch, FH), jnp.float32),
            pltpu.VMEM_SHARED((acc_rows, FH), jnp.float32),
            pltpu.SemaphoreType.DMA,
            pltpu.SemaphoreType.DMA,
        ],
    )
    def agg(h0, h1, srcs, dsts, zeros, o0, o1,
            src_v, dst_v, rows0, rows1, acc, sem0, sem1):
        c = lax.axis_index("c")
        s = lax.axis_index("s")
        pltpu.sync_copy(zeros.at[pl.ds(0, out_rpt)],
                        acc.at[pl.ds(s * out_rpt, out_rpt)])
        pltpu.sync_copy(srcs.at[s], src_v)
        pltpu.sync_copy(dsts.at[s], dst_v)
        plsc.subcore_barrier()

        def run(h, o):
            def gather(j, buf, sem):
                return pltpu.make_async_copy(h.at[src_v.at[j]], buf, sem)

            gather(0, rows0, sem0).start()
            if nch > 1:
                gather(1, rows1, sem1).start()

            def body(i, carry):
                j = i * 2
                gather(j, rows0, sem0).wait()
                pltpu.sync_copy(rows0, acc.at[dst_v.at[j]], add=True)

                @pl.when(j + 2 < nch)
                def _():
                    gather(j + 2, rows0, sem0).start()

                gather(j + 1, rows1, sem1).wait()
                pltpu.sync_copy(rows1, acc.at[dst_v.at[j + 1]], add=True)

                @pl.when(j + 3 < nch)
                def _():
                    gather(j + 3, rows1, sem1).start()

                return carry

            lax.fori_loop(0, nch // 2, body, 0)
            if nch % 2:
                gather(nch - 1, rows0, sem0).wait()
                pltpu.sync_copy(rows0, acc.at[dst_v.at[nch - 1]], add=True)
            plsc.subcore_barrier()
            pltpu.sync_copy(acc.at[pl.ds(s * out_rpt, out_rpt)],
                            o.at[pl.ds(s * out_rpt, out_rpt)])

        @pl.when(c == 0)
        def _():
            run(h0, o0)

        @pl.when(c == 1)
        def _():
            run(h1, o1)

    return agg


def _make_counts():
    mesh = plsc.VectorSubcoreMesh(core_axis_name="c", subcore_axis_name="s")
    out_sds = jax.ShapeDtypeStruct((NACC, FH), jnp.float32)

    @functools.partial(
        pl.kernel,
        out_type=[out_sds, out_sds],
        mesh=mesh,
        scratch_types=[
            pltpu.VMEM((CCH, CH), jnp.int32),
            pltpu.VMEM((CH, FH), jnp.float32),
            pltpu.VMEM_SHARED((NACC, FH), jnp.float32),
            pltpu.SemaphoreType.DMA,
        ],
    )
    def counts(idx_all, ones, zeros, o0, o1, idx_v, rows_v, acc, sem):
        c = lax.axis_index("c")
        s = lax.axis_index("s")
        w = c * NTILE + s
        pltpu.sync_copy(zeros.at[pl.ds(0, CRPT)],
                        acc.at[pl.ds(s * CRPT, CRPT)])
        pltpu.sync_copy(idx_all.at[w], idx_v)
        pltpu.sync_copy(ones, rows_v)
        plsc.subcore_barrier()

        def scat(j):
            return pltpu.make_async_copy(rows_v, acc.at[idx_v.at[j]], sem)

        def body(j, carry):
            scat(j).start(add=True)

            @pl.when(j >= 8)
            def _():
                scat(j).wait()

            return carry

        lax.fori_loop(0, CCH, body, 0)
        for _ in range(8):
            scat(0).wait()
        plsc.subcore_barrier()

        def out(o):
            pltpu.sync_copy(acc.at[pl.ds(s * CRPT, CRPT)],
                            o.at[pl.ds(s * CRPT, CRPT)])

        @pl.when(c == 0)
        def _():
            out(o0)

        @pl.when(c == 1)
        def _():
            out(o1)

    return counts


def _layer_body(mean, a0, a1, h0, h1, wrel, wroot, b, c0, c1, o0, o1):
    x0 = a0[...]
    x1 = a1[...]
    if mean:
        scale = 1.0 / jnp.maximum(c0[:, :1] + c1[:, :1], 1.0)
        x0 = x0 * scale
        x1 = x1 * scale
    p = (jnp.dot(x0, wrel[:FH, :], preferred_element_type=jnp.float32)
         + jnp.dot(x1, wrel[FH:, :], preferred_element_type=jnp.float32)
         + jnp.dot(h0[...], wroot[:FH, :], preferred_element_type=jnp.float32)
         + jnp.dot(h1[...], wroot[FH:, :], preferred_element_type=jnp.float32)
         + b[...])
    r = jnp.maximum(p, 0.0)
    o0[...] = r[:, :FH]
    o1[...] = r[:, FH:]


def _make_layer(mean):
    return pl.pallas_call(
        functools.partial(_layer_body, mean),
        grid=(NP // RB,),
        in_specs=[pl.BlockSpec((RB, FH), lambda i: (i, 0))] * 4 + [
            pl.BlockSpec((F, F), lambda i: (0, 0)),
            pl.BlockSpec((F, F), lambda i: (0, 0)),
            pl.BlockSpec((1, F), lambda i: (0, 0)),
            pl.BlockSpec((RB, 8), lambda i: (i, 0)),
            pl.BlockSpec((RB, 8), lambda i: (i, 0)),
        ],
        out_specs=[pl.BlockSpec((RB, FH), lambda i: (i, 0))] * 2,
        out_shape=[jax.ShapeDtypeStruct((NP, FH), jnp.float32)] * 2,
    )


_layer_add = _make_layer(False)
_layer_mean = _make_layer(True)


def _final_body(p0, p1, g0, g1, w, b, o):
    scale = 1.0 / jnp.maximum(g0[:, :1] + g1[:, :1], 1.0)
    pooled = jnp.concatenate([p0[:G, :] * scale, p1[:G, :] * scale], axis=1)
    o[...] = jnp.dot(pooled, w[...], preferred_element_type=jnp.float32) + b[...]


_final = pl.pallas_call(
    _final_body,
    grid=(1,),
    in_specs=[
        pl.BlockSpec((GP, FH), lambda i: (0, 0)),
        pl.BlockSpec((GP, FH), lambda i: (0, 0)),
        pl.BlockSpec((G, 8), lambda i: (0, 0)),
        pl.BlockSpec((G, 8), lambda i: (0, 0)),
        pl.BlockSpec((F, OUT), lambda i: (0, 0)),
        pl.BlockSpec((1, OUT), lambda i: (0, 0)),
    ],
    out_specs=pl.BlockSpec((G, OUT), lambda i: (0, 0)),
    out_shape=jax.ShapeDtypeStruct((G, OUT), jnp.float32),
)


def kernel(x, edge_index, batch, W_rel1, b_rel1, W_root1, W_rel2, b_rel2,
           W_root2, W_rel3, b_rel3, W_root3, W_out, b_out):
    src = edge_index[0].astype(jnp.int32)
    dst = edge_index[1].astype(jnp.int32)
    ar = jnp.arange(EPAD - E, dtype=jnp.int32)
    srcs = jnp.concatenate([src, ar % N]).reshape(NTILE, ECH, CHE)
    dsts = jnp.concatenate([dst, N + ar % (NP - N)]).reshape(NTILE, ECH, CHE)

    b32 = batch.astype(jnp.int32)
    arp = jnp.arange(PPAD - N, dtype=jnp.int32)
    prow = jnp.concatenate(
        [jnp.arange(N, dtype=jnp.int32), N + arp % (NP - N)]
    ).reshape(NTILE, PCH, CH)
    pgrp = jnp.concatenate([b32, G + arp % (GP - G)]).reshape(NTILE, PCH, CH)

    arc = jnp.arange(CPAD - EPAD - N, dtype=jnp.int32)
    cidx = jnp.concatenate([
        dst, N + ar % (NP - N),
        NP + b32,
        NP + G + arc % (GP - G),
    ]).reshape(32, CCH, CH)

    zeros = jnp.zeros((CRPT, FH), jnp.float32)
    ones = jnp.ones((CH, FH), jnp.float32)
    xp = jnp.pad(x, ((0, NP - N), (0, 0)))
    x0 = xp[:, :FH]
    x1 = xp[:, FH:]

    agg_e = _make_agg_deep(ECH, CHE, NP, RPT)
    agg_p = _make_agg(PCH, CH, GP, GRPT)
    cnt = _make_counts()

    cnt0, cnt1 = cnt(cidx, ones, zeros)
    deg0 = cnt0[:NP, :8]
    deg1 = cnt1[:NP, :8]
    gc0 = cnt0[NP:NP + G, :8]
    gc1 = cnt1[NP:NP + G, :8]

    A0, A1 = agg_e(x0, x1, srcs, dsts, zeros)
    h0, h1 = _layer_add(A0, A1, x0, x1, W_rel1, W_root1,
                        b_rel1.reshape(1, -1), deg0, deg1)
    A0, A1 = agg_e(h0, h1, srcs, dsts, zeros)
    h0, h1 = _layer_mean(A0, A1, h0, h1, W_rel2, W_root2,
                         b_rel2.reshape(1, -1), deg0, deg1)
    A0, A1 = agg_e(h0, h1, srcs, dsts, zeros)
    h0, h1 = _layer_mean(A0, A1, h0, h1, W_rel3, W_root3,
                         b_rel3.reshape(1, -1), deg0, deg1)
    P0, P1 = agg_p(h0, h1, prow, pgrp, zeros)
    return _final(P0, P1, gc0, gc1, W_out, b_out.reshape(1, -1))


# speedup vs baseline: 1.0418x; 1.0418x over previous
import functools

import jax
import jax.numpy as jnp
from jax import lax
from jax.experimental import pallas as pl
from jax.experimental.pallas import tpu as pltpu
from jax.experimental.pallas import tpu_sc as plsc

N = 10000
E = 160000
F = 256
G = 64
OUT = 24
FH = 128
NTILE = 16
NP = 10112
RPT = NP // NTILE
CH = 64
CHE = 64
ECH = 168
EPAD = NTILE * ECH * CHE
GRPC = 24
NB = 3
PCH = 10
PPAD = NTILE * PCH * CH
GP = 128
GRPT = GP // NTILE
RB = 2528

NACC = NP + GP
CRPT = NACC // NTILE
CCH = 89
CPAD = 32 * CCH * CH


def _make_agg_deep(nch, ch, acc_rows, out_rpt):
    mesh = plsc.VectorSubcoreMesh(core_axis_name="c", subcore_axis_name="s")
    out_sds = jax.ShapeDtypeStruct((acc_rows, FH), jnp.float32)
    ngrp = nch // GRPC

    @functools.partial(
        pl.kernel,
        out_type=[out_sds, out_sds],
        mesh=mesh,
        scratch_types=[pltpu.VMEM((GRPC, ch), jnp.int32)] * 4
        + [pltpu.VMEM((ch, FH), jnp.float32)] * NB
        + [pltpu.VMEM_SHARED((acc_rows, FH), jnp.float32)]
        + [pltpu.SemaphoreType.DMA] * (2 * NB + 2),
    )
    def agg(h0, h1, srcs, dsts, zeros, o0, o1,
            sv0, dv0, sv1, dv1, r0, r1, r2, acc,
            g0, g1, g2, s0, s1, s2, i0, i1):
        rows = [r0, r1, r2]
        gsem = [g0, g1, g2]
        ssem = [s0, s1, s2]
        svs = [sv0, sv1]
        dvs = [dv0, dv1]
        c = lax.axis_index("c")
        s = lax.axis_index("s")
        pltpu.sync_copy(zeros.at[pl.ds(0, out_rpt)],
                        acc.at[pl.ds(s * out_rpt, out_rpt)])
        plsc.subcore_barrier()

        def run(h, o):
            def istage(g, p):
                return (
                    pltpu.make_async_copy(
                        srcs.at[s, pl.ds(g * GRPC, GRPC)], svs[p], i0),
                    pltpu.make_async_copy(
                        dsts.at[s, pl.ds(g * GRPC, GRPC)], dvs[p], i1),
                )

            for cp in istage(0, 0):
                cp.start()

            for g in range(ngrp):
                p = g % 2
                src_v = svs[p]
                dst_v = dvs[p]
                for cp in istage(g, p):
                    cp.wait()
                if g + 1 < ngrp:
                    for cp in istage(g + 1, 1 - p):
                        cp.start()

                def gcopy(j, b):
                    return pltpu.make_async_copy(h.at[src_v.at[j]], rows[b],
                                                 gsem[b])

                def scopy(j, b):
                    return pltpu.make_async_copy(rows[b], acc.at[dst_v.at[j]],
                                                 ssem[b])

                gcopy(0, 0).start()
                gcopy(1, 1).start()

                def body(i, carry):
                    for b in range(NB):
                        j = i * NB + b
                        gcopy(j, b).wait()
                        scopy(j, b).start(add=True)
                        bn = (b + 2) % NB

                        @pl.when((j + 2 < GRPC) & (j >= 1))
                        def _():
                            scopy(j, bn).wait()

                        @pl.when(j + 2 < GRPC)
                        def _():
                            gcopy(j + 2, bn).start()

                    return carry

                lax.fori_loop(0, GRPC // NB, body, 0)
                for b in range(NB):
                    scopy(0, b).wait()

            plsc.subcore_barrier()
            pltpu.sync_copy(acc.at[pl.ds(s * out_rpt, out_rpt)],
                            o.at[pl.ds(s * out_rpt, out_rpt)])

        @pl.when(c == 0)
        def _():
            run(h0, o0)

        @pl.when(c == 1)
        def _():
            run(h1, o1)

    return agg


def _make_agg(nch, ch, acc_rows, out_rpt):
    mesh = plsc.VectorSubcoreMesh(core_axis_name="c", subcore_axis_name="s")
    out_sds = jax.ShapeDtypeStruct((acc_rows, FH), jnp.float32)

    @functools.partial(
        pl.kernel,
        out_type=[out_sds, out_sds],
        mesh=mesh,
        scratch_types=[
            pltpu.VMEM((nch, ch), jnp.int32),
            pltpu.VMEM((nch, ch), jnp.int32),
            pltpu.VMEM((ch, FH), jnp.float32),
            pltpu.VMEM((ch, FH), jnp.float32),
            pltpu.VMEM_SHARED((acc_rows, FH), jnp.float32),
            pltpu.SemaphoreType.DMA,
            pltpu.SemaphoreType.DMA,
        ],
    )
    def agg(h0, h1, srcs, dsts, zeros, o0, o1,
            src_v, dst_v, rows0, rows1, acc, sem0, sem1):
        c = lax.axis_index("c")
        s = lax.axis_index("s")
        pltpu.sync_copy(zeros.at[pl.ds(0, out_rpt)],
                        acc.at[pl.ds(s * out_rpt, out_rpt)])
        pltpu.sync_copy(srcs.at[s], src_v)
        pltpu.sync_copy(dsts.at[s], dst_v)
        plsc.subcore_barrier()

        def run(h, o):
            def gather(j, buf, sem):
                return pltpu.make_async_copy(h.at[src_v.at[j]], buf, sem)

            gather(0, rows0, sem0).start()
            if nch > 1:
                gather(1, rows1, sem1).start()

            def body(i, carry):
                j = i * 2
                gather(j, rows0, sem0).wait()
                pltpu.sync_copy(rows0, acc.at[dst_v.at[j]], add=True)

                @pl.when(j + 2 < nch)
                def _():
                    gather(j + 2, rows0, sem0).start()

                gather(j + 1, rows1, sem1).wait()
                pltpu.sync_copy(rows1, acc.at[dst_v.at[j + 1]], add=True)

                @pl.when(j + 3 < nch)
                def _():
                    gather(j + 3, rows1, sem1).start()

                return carry

            lax.fori_loop(0, nch // 2, body, 0)
            if nch % 2:
                gather(nch - 1, rows0, sem0).wait()
                pltpu.sync_copy(rows0, acc.at[dst_v.at[nch - 1]], add=True)
            plsc.subcore_barrier()
            pltpu.sync_copy(acc.at[pl.ds(s * out_rpt, out_rpt)],
                            o.at[pl.ds(s * out_rpt, out_rpt)])

        @pl.when(c == 0)
        def _():
            run(h0, o0)

        @pl.when(c == 1)
        def _():
            run(h1, o1)

    return agg


def _make_counts():
    mesh = plsc.VectorSubcoreMesh(core_axis_name="c", subcore_axis_name="s")
    out_sds = jax.ShapeDtypeStruct((NACC, FH), jnp.float32)

    @functools.partial(
        pl.kernel,
        out_type=[out_sds, out_sds],
        mesh=mesh,
        scratch_types=[
            pltpu.VMEM((CCH, CH), jnp.int32),
            pltpu.VMEM((CH, FH), jnp.float32),
            pltpu.VMEM_SHARED((NACC, FH), jnp.float32),
        ],
    )
    def counts(idx_all, ones, zeros, o0, o1, idx_v, rows_v, acc):
        c = lax.axis_index("c")
        s = lax.axis_index("s")
        w = c * NTILE + s
        pltpu.sync_copy(zeros.at[pl.ds(0, CRPT)],
                        acc.at[pl.ds(s * CRPT, CRPT)])
        pltpu.sync_copy(idx_all.at[w], idx_v)
        pltpu.sync_copy(ones, rows_v)
        plsc.subcore_barrier()

        def body(j, carry):
            pltpu.sync_copy(rows_v, acc.at[idx_v.at[j]], add=True)
            return carry

        lax.fori_loop(0, CCH, body, 0)
        plsc.subcore_barrier()

        def out(o):
            pltpu.sync_copy(acc.at[pl.ds(s * CRPT, CRPT)],
                            o.at[pl.ds(s * CRPT, CRPT)])

        @pl.when(c == 0)
        def _():
            out(o0)

        @pl.when(c == 1)
        def _():
            out(o1)

    return counts


def _layer_body(mean, a0, a1, h0, h1, wrel, wroot, b, c0, c1, o0, o1):
    x0 = a0[...]
    x1 = a1[...]
    if mean:
        scale = 1.0 / jnp.maximum(c0[:, :1] + c1[:, :1], 1.0)
        x0 = x0 * scale
        x1 = x1 * scale
    p = (jnp.dot(x0, wrel[:FH, :], preferred_element_type=jnp.float32)
         + jnp.dot(x1, wrel[FH:, :], preferred_element_type=jnp.float32)
         + jnp.dot(h0[...], wroot[:FH, :], preferred_element_type=jnp.float32)
         + jnp.dot(h1[...], wroot[FH:, :], preferred_element_type=jnp.float32)
         + b[...])
    r = jnp.maximum(p, 0.0)
    o0[...] = r[:, :FH]
    o1[...] = r[:, FH:]


def _make_layer(mean):
    return pl.pallas_call(
        functools.partial(_layer_body, mean),
        grid=(NP // RB,),
        in_specs=[pl.BlockSpec((RB, FH), lambda i: (i, 0))] * 4 + [
            pl.BlockSpec((F, F), lambda i: (0, 0)),
            pl.BlockSpec((F, F), lambda i: (0, 0)),
            pl.BlockSpec((1, F), lambda i: (0, 0)),
            pl.BlockSpec((RB, 8), lambda i: (i, 0)),
            pl.BlockSpec((RB, 8), lambda i: (i, 0)),
        ],
        out_specs=[pl.BlockSpec((RB, FH), lambda i: (i, 0))] * 2,
        out_shape=[jax.ShapeDtypeStruct((NP, FH), jnp.float32)] * 2,
    )


_layer_add = _make_layer(False)
_layer_mean = _make_layer(True)


def _final_body(p0, p1, g0, g1, w, b, o):
    scale = 1.0 / jnp.maximum(g0[:, :1] + g1[:, :1], 1.0)
    pooled = jnp.concatenate([p0[:G, :] * scale, p1[:G, :] * scale], axis=1)
    o[...] = jnp.dot(pooled, w[...], preferred_element_type=jnp.float32) + b[...]


_final = pl.pallas_call(
    _final_body,
    grid=(1,),
    in_specs=[
        pl.BlockSpec((GP, FH), lambda i: (0, 0)),
        pl.BlockSpec((GP, FH), lambda i: (0, 0)),
        pl.BlockSpec((G, 8), lambda i: (0, 0)),
        pl.BlockSpec((G, 8), lambda i: (0, 0)),
        pl.BlockSpec((F, OUT), lambda i: (0, 0)),
        pl.BlockSpec((1, OUT), lambda i: (0, 0)),
    ],
    out_specs=pl.BlockSpec((G, OUT), lambda i: (0, 0)),
    out_shape=jax.ShapeDtypeStruct((G, OUT), jnp.float32),
)


def kernel(x, edge_index, batch, W_rel1, b_rel1, W_root1, W_rel2, b_rel2,
           W_root2, W_rel3, b_rel3, W_root3, W_out, b_out):
    src = edge_index[0].astype(jnp.int32)
    dst = edge_index[1].astype(jnp.int32)
    ar = jnp.arange(EPAD - E, dtype=jnp.int32)
    srcs = jnp.concatenate([src, ar % N]).reshape(NTILE, ECH, CHE)
    dsts = jnp.concatenate([dst, N + ar % (NP - N)]).reshape(NTILE, ECH, CHE)

    b32 = batch.astype(jnp.int32)
    arp = jnp.arange(PPAD - N, dtype=jnp.int32)
    prow = jnp.concatenate(
        [jnp.arange(N, dtype=jnp.int32), N + arp % (NP - N)]
    ).reshape(NTILE, PCH, CH)
    pgrp = jnp.concatenate([b32, G + arp % (GP - G)]).reshape(NTILE, PCH, CH)

    arc = jnp.arange(CPAD - EPAD - N, dtype=jnp.int32)
    cidx = jnp.concatenate([
        dst, N + ar % (NP - N),
        NP + b32,
        NP + G + arc % (GP - G),
    ]).reshape(32, CCH, CH)

    zeros = jnp.zeros((CRPT, FH), jnp.float32)
    ones = jnp.ones((CH, FH), jnp.float32)
    xp = jnp.pad(x, ((0, NP - N), (0, 0)))
    x0 = xp[:, :FH]
    x1 = xp[:, FH:]

    agg_e = _make_agg_deep(ECH, CHE, NP, RPT)
    agg_p = _make_agg(PCH, CH, GP, GRPT)
    cnt = _make_counts()

    cnt0, cnt1 = cnt(cidx, ones, zeros)
    deg0 = cnt0[:NP, :8]
    deg1 = cnt1[:NP, :8]
    gc0 = cnt0[NP:NP + G, :8]
    gc1 = cnt1[NP:NP + G, :8]

    A0, A1 = agg_e(x0, x1, srcs, dsts, zeros)
    h0, h1 = _layer_add(A0, A1, x0, x1, W_rel1, W_root1,
                        b_rel1.reshape(1, -1), deg0, deg1)
    A0, A1 = agg_e(h0, h1, srcs, dsts, zeros)
    h0, h1 = _layer_mean(A0, A1, h0, h1, W_rel2, W_root2,
                         b_rel2.reshape(1, -1), deg0, deg1)
    A0, A1 = agg_e(h0, h1, srcs, dsts, zeros)
    h0, h1 = _layer_mean(A0, A1, h0, h1, W_rel3, W_root3,
                         b_rel3.reshape(1, -1), deg0, deg1)
    P0, P1 = agg_p(h0, h1, prow, pgrp, zeros)
    return _final(P0, P1, gc0, gc1, W_out, b_out.reshape(1, -1))


# speedup vs baseline: 1.0561x; 1.0138x over previous
import functools

import jax
import jax.numpy as jnp
from jax import lax
from jax.experimental import pallas as pl
from jax.experimental.pallas import tpu as pltpu
from jax.experimental.pallas import tpu_sc as plsc

N = 10000
E = 160000
F = 256
G = 64
OUT = 24
FH = 128
NTILE = 16
NP = 10112
RPT = NP // NTILE
CH = 64
CHE = 64
ECH = 168
EPAD = NTILE * ECH * CHE
GRPC = 24
NB = 3
PCH = 10
PPAD = NTILE * PCH * CH
GP = 128
GRPT = GP // NTILE
RB = 2528

NACC = NP + GP
CRPT = NACC // NTILE
CCH = 89
CPAD = 32 * CCH * CH


def _make_agg_deep(nch, ch, acc_rows, out_rpt):
    mesh = plsc.VectorSubcoreMesh(core_axis_name="c", subcore_axis_name="s")
    out_sds = jax.ShapeDtypeStruct((acc_rows, FH), jnp.float32)
    ngrp = nch // GRPC

    @functools.partial(
        pl.kernel,
        out_type=[out_sds, out_sds],
        mesh=mesh,
        scratch_types=[pltpu.VMEM((GRPC, ch), jnp.int32)] * 4
        + [pltpu.VMEM((ch, FH), jnp.float32)] * NB
        + [pltpu.VMEM_SHARED((acc_rows, FH), jnp.float32)]
        + [pltpu.SemaphoreType.DMA] * (2 * NB + 2),
    )
    def agg(h0, h1, srcs, dsts, zeros, o0, o1,
            sv0, dv0, sv1, dv1, r0, r1, r2, acc,
            g0, g1, g2, s0, s1, s2, i0, i1):
        rows = [r0, r1, r2]
        gsem = [g0, g1, g2]
        ssem = [s0, s1, s2]
        svs = [sv0, sv1]
        dvs = [dv0, dv1]
        c = lax.axis_index("c")
        s = lax.axis_index("s")
        pltpu.sync_copy(zeros.at[pl.ds(0, out_rpt)],
                        acc.at[pl.ds(s * out_rpt, out_rpt)])
        plsc.subcore_barrier()

        def run(h, o):
            def istage(g, p):
                return (
                    pltpu.make_async_copy(
                        srcs.at[s, pl.ds(g * GRPC, GRPC)], svs[p], i0),
                    pltpu.make_async_copy(
                        dsts.at[s, pl.ds(g * GRPC, GRPC)], dvs[p], i1),
                )

            for cp in istage(0, 0):
                cp.start()
            if ngrp > 1:
                for cp in istage(1, 1):
                    cp.start()

            for g in range(ngrp):
                p = g % 2
                src_v = svs[p]
                dst_v = dvs[p]

                def gcopy(j, b, sv=src_v):
                    return pltpu.make_async_copy(h.at[sv.at[j]], rows[b],
                                                 gsem[b])

                def scopy(j, b, dv=dst_v):
                    return pltpu.make_async_copy(rows[b], acc.at[dv.at[j]],
                                                 ssem[b])

                if g == 0:
                    for cp in istage(0, 0):
                        cp.wait()
                    gcopy(0, 0).start()
                    gcopy(1, 1).start()
                else:
                    scopy(0, 2).wait()
                    if g + 1 < ngrp:
                        for cp in istage(g + 1, 1 - p):
                            cp.start()

                def body(i, carry):
                    for b in range(NB):
                        j = i * NB + b
                        gcopy(j, b).wait()
                        scopy(j, b).start(add=True)
                        bn = (b + 2) % NB

                        @pl.when((j + 2 < GRPC) & (j >= 1))
                        def _():
                            scopy(j, bn).wait()

                        @pl.when(j + 2 < GRPC)
                        def _():
                            gcopy(j + 2, bn).start()

                    return carry

                lax.fori_loop(0, GRPC // NB, body, 0)
                if g + 1 < ngrp:
                    svn = svs[1 - p]
                    for cp in istage(g + 1, 1 - p):
                        cp.wait()
                    scopy(0, 0).wait()
                    pltpu.make_async_copy(h.at[svn.at[0]], rows[0],
                                          gsem[0]).start()
                    scopy(0, 1).wait()
                    pltpu.make_async_copy(h.at[svn.at[1]], rows[1],
                                          gsem[1]).start()
                else:
                    for b in range(NB):
                        scopy(0, b).wait()

            plsc.subcore_barrier()
            pltpu.sync_copy(acc.at[pl.ds(s * out_rpt, out_rpt)],
                            o.at[pl.ds(s * out_rpt, out_rpt)])

        @pl.when(c == 0)
        def _():
            run(h0, o0)

        @pl.when(c == 1)
        def _():
            run(h1, o1)

    return agg


def _make_agg(nch, ch, acc_rows, out_rpt):
    mesh = plsc.VectorSubcoreMesh(core_axis_name="c", subcore_axis_name="s")
    out_sds = jax.ShapeDtypeStruct((acc_rows, FH), jnp.float32)

    @functools.partial(
        pl.kernel,
        out_type=[out_sds, out_sds],
        mesh=mesh,
        scratch_types=[
            pltpu.VMEM((nch, ch), jnp.int32),
            pltpu.VMEM((nch, ch), jnp.int32),
            pltpu.VMEM((ch, FH), jnp.float32),
            pltpu.VMEM((ch, FH), jnp.float32),
            pltpu.VMEM_SHARED((acc_rows, FH), jnp.float32),
            pltpu.SemaphoreType.DMA,
            pltpu.SemaphoreType.DMA,
        ],
    )
    def agg(h0, h1, srcs, dsts, zeros, o0, o1,
            src_v, dst_v, rows0, rows1, acc, sem0, sem1):
        c = lax.axis_index("c")
        s = lax.axis_index("s")
        pltpu.sync_copy(zeros.at[pl.ds(0, out_rpt)],
                        acc.at[pl.ds(s * out_rpt, out_rpt)])
        pltpu.sync_copy(srcs.at[s], src_v)
        pltpu.sync_copy(dsts.at[s], dst_v)
        plsc.subcore_barrier()

        def run(h, o):
            def gather(j, buf, sem):
                return pltpu.make_async_copy(h.at[src_v.at[j]], buf, sem)

            gather(0, rows0, sem0).start()
            if nch > 1:
                gather(1, rows1, sem1).start()

            def body(i, carry):
                j = i * 2
                gather(j, rows0, sem0).wait()
                pltpu.sync_copy(rows0, acc.at[dst_v.at[j]], add=True)

                @pl.when(j + 2 < nch)
                def _():
                    gather(j + 2, rows0, sem0).start()

                gather(j + 1, rows1, sem1).wait()
                pltpu.sync_copy(rows1, acc.at[dst_v.at[j + 1]], add=True)

                @pl.when(j + 3 < nch)
                def _():
                    gather(j + 3, rows1, sem1).start()

                return carry

            lax.fori_loop(0, nch // 2, body, 0)
            if nch % 2:
                gather(nch - 1, rows0, sem0).wait()
                pltpu.sync_copy(rows0, acc.at[dst_v.at[nch - 1]], add=True)
            plsc.subcore_barrier()
            pltpu.sync_copy(acc.at[pl.ds(s * out_rpt, out_rpt)],
                            o.at[pl.ds(s * out_rpt, out_rpt)])

        @pl.when(c == 0)
        def _():
            run(h0, o0)

        @pl.when(c == 1)
        def _():
            run(h1, o1)

    return agg


def _make_counts():
    mesh = plsc.VectorSubcoreMesh(core_axis_name="c", subcore_axis_name="s")
    out_sds = jax.ShapeDtypeStruct((NACC, FH), jnp.float32)

    @functools.partial(
        pl.kernel,
        out_type=[out_sds, out_sds],
        mesh=mesh,
        scratch_types=[
            pltpu.VMEM((CCH, CH), jnp.int32),
            pltpu.VMEM((CH, FH), jnp.float32),
            pltpu.VMEM_SHARED((NACC, FH), jnp.float32),
        ],
    )
    def counts(idx_all, ones, zeros, o0, o1, idx_v, rows_v, acc):
        c = lax.axis_index("c")
        s = lax.axis_index("s")
        w = c * NTILE + s
        pltpu.sync_copy(zeros.at[pl.ds(0, CRPT)],
                        acc.at[pl.ds(s * CRPT, CRPT)])
        pltpu.sync_copy(idx_all.at[w], idx_v)
        pltpu.sync_copy(ones, rows_v)
        plsc.subcore_barrier()

        def body(j, carry):
            pltpu.sync_copy(rows_v, acc.at[idx_v.at[j]], add=True)
            return carry

        lax.fori_loop(0, CCH, body, 0)
        plsc.subcore_barrier()

        def out(o):
            pltpu.sync_copy(acc.at[pl.ds(s * CRPT, CRPT)],
                            o.at[pl.ds(s * CRPT, CRPT)])

        @pl.when(c == 0)
        def _():
            out(o0)

        @pl.when(c == 1)
        def _():
            out(o1)

    return counts


def _layer_body(mean, a0, a1, h0, h1, wrel, wroot, b, c0, c1, o0, o1):
    x0 = a0[...]
    x1 = a1[...]
    if mean:
        scale = 1.0 / jnp.maximum(c0[:, :1] + c1[:, :1], 1.0)
        x0 = x0 * scale
        x1 = x1 * scale
    p = (jnp.dot(x0, wrel[:FH, :], preferred_element_type=jnp.float32)
         + jnp.dot(x1, wrel[FH:, :], preferred_element_type=jnp.float32)
         + jnp.dot(h0[...], wroot[:FH, :], preferred_element_type=jnp.float32)
         + jnp.dot(h1[...], wroot[FH:, :], preferred_element_type=jnp.float32)
         + b[...])
    r = jnp.maximum(p, 0.0)
    o0[...] = r[:, :FH]
    o1[...] = r[:, FH:]


def _make_layer(mean):
    return pl.pallas_call(
        functools.partial(_layer_body, mean),
        grid=(NP // RB,),
        in_specs=[pl.BlockSpec((RB, FH), lambda i: (i, 0))] * 4 + [
            pl.BlockSpec((F, F), lambda i: (0, 0)),
            pl.BlockSpec((F, F), lambda i: (0, 0)),
            pl.BlockSpec((1, F), lambda i: (0, 0)),
            pl.BlockSpec((RB, 8), lambda i: (i, 0)),
            pl.BlockSpec((RB, 8), lambda i: (i, 0)),
        ],
        out_specs=[pl.BlockSpec((RB, FH), lambda i: (i, 0))] * 2,
        out_shape=[jax.ShapeDtypeStruct((NP, FH), jnp.float32)] * 2,
    )


_layer_add = _make_layer(False)
_layer_mean = _make_layer(True)


def _final_body(p0, p1, g0, g1, w, b, o):
    scale = 1.0 / jnp.maximum(g0[:, :1] + g1[:, :1], 1.0)
    pooled = jnp.concatenate([p0[:G, :] * scale, p1[:G, :] * scale], axis=1)
    o[...] = jnp.dot(pooled, w[...], preferred_element_type=jnp.float32) + b[...]


_final = pl.pallas_call(
    _final_body,
    grid=(1,),
    in_specs=[
        pl.BlockSpec((GP, FH), lambda i: (0, 0)),
        pl.BlockSpec((GP, FH), lambda i: (0, 0)),
        pl.BlockSpec((G, 8), lambda i: (0, 0)),
        pl.BlockSpec((G, 8), lambda i: (0, 0)),
        pl.BlockSpec((F, OUT), lambda i: (0, 0)),
        pl.BlockSpec((1, OUT), lambda i: (0, 0)),
    ],
    out_specs=pl.BlockSpec((G, OUT), lambda i: (0, 0)),
    out_shape=jax.ShapeDtypeStruct((G, OUT), jnp.float32),
)


def kernel(x, edge_index, batch, W_rel1, b_rel1, W_root1, W_rel2, b_rel2,
           W_root2, W_rel3, b_rel3, W_root3, W_out, b_out):
    src = edge_index[0].astype(jnp.int32)
    dst = edge_index[1].astype(jnp.int32)
    ar = jnp.arange(EPAD - E, dtype=jnp.int32)
    srcs = jnp.concatenate([src, ar % N]).reshape(NTILE, ECH, CHE)
    dsts = jnp.concatenate([dst, N + ar % (NP - N)]).reshape(NTILE, ECH, CHE)

    b32 = batch.astype(jnp.int32)
    arp = jnp.arange(PPAD - N, dtype=jnp.int32)
    prow = jnp.concatenate(
        [jnp.arange(N, dtype=jnp.int32), N + arp % (NP - N)]
    ).reshape(NTILE, PCH, CH)
    pgrp = jnp.concatenate([b32, G + arp % (GP - G)]).reshape(NTILE, PCH, CH)

    arc = jnp.arange(CPAD - EPAD - N, dtype=jnp.int32)
    cidx = jnp.concatenate([
        dst, N + ar % (NP - N),
        NP + b32,
        NP + G + arc % (GP - G),
    ]).reshape(32, CCH, CH)

    zeros = jnp.zeros((CRPT, FH), jnp.float32)
    ones = jnp.ones((CH, FH), jnp.float32)
    xp = jnp.pad(x, ((0, NP - N), (0, 0)))
    x0 = xp[:, :FH]
    x1 = xp[:, FH:]

    agg_e = _make_agg_deep(ECH, CHE, NP, RPT)
    agg_p = _make_agg(PCH, CH, GP, GRPT)
    cnt = _make_counts()

    cnt0, cnt1 = cnt(cidx, ones, zeros)
    deg0 = cnt0[:NP, :8]
    deg1 = cnt1[:NP, :8]
    gc0 = cnt0[NP:NP + G, :8]
    gc1 = cnt1[NP:NP + G, :8]

    A0, A1 = agg_e(x0, x1, srcs, dsts, zeros)
    h0, h1 = _layer_add(A0, A1, x0, x1, W_rel1, W_root1,
                        b_rel1.reshape(1, -1), deg0, deg1)
    A0, A1 = agg_e(h0, h1, srcs, dsts, zeros)
    h0, h1 = _layer_mean(A0, A1, h0, h1, W_rel2, W_root2,
                         b_rel2.reshape(1, -1), deg0, deg1)
    A0, A1 = agg_e(h0, h1, srcs, dsts, zeros)
    h0, h1 = _layer_mean(A0, A1, h0, h1, W_rel3, W_root3,
                         b_rel3.reshape(1, -1), deg0, deg1)
    P0, P1 = agg_p(h0, h1, prow, pgrp, zeros)
    return _final(P0, P1, gc0, gc1, W_out, b_out.reshape(1, -1))
